# Initial kernel scaffold; baseline (speedup 1.0000x reference)
import jax
import jax.numpy as jnp
from jax.experimental import pallas as pl


def kernel(hidden_states, cls_index):
    raise NotImplementedError("write your pallas kernel here")


# speedup vs baseline: 4.0510x; 4.0510x over previous
import functools

import jax
import jax.numpy as jnp
from jax import lax
from jax.experimental import pallas as pl
from jax.experimental.pallas import tpu as pltpu
from jax.experimental.pallas import tpu_sc as plsc


def kernel(hidden_states, cls_index):
    B, S, H = hidden_states.shape
    half = B // 2
    nout = 2 * B
    table = hidden_states.reshape(B * S, H)
    cls16 = jnp.zeros((16,), jnp.int32).at[:B].set(cls_index)

    mesh = plsc.VectorSubcoreMesh(core_axis_name="c", subcore_axis_name="s")

    @functools.partial(
        pl.kernel,
        out_type=jax.ShapeDtypeStruct((nout, H), jnp.float32),
        mesh=mesh,
        scratch_types=[
            pltpu.VMEM((16,), jnp.int32),
            pltpu.VMEM((16,), jnp.int32),
            pltpu.VMEM((nout, H), jnp.float32),
            pltpu.SemaphoreType.DMA,
        ],
    )
    def gather_rows(table_hbm, cls_hbm, out_hbm, cls_v, idx_v, rows_v, sem):
        cid = lax.axis_index("c")
        sid = lax.axis_index("s")

        @pl.when(jnp.logical_and(cid == 0, sid == 0))
        def _():
            pltpu.sync_copy(cls_hbm, cls_v)
            lanes = lax.iota(jnp.int32, 16)
            j = lanes & (B - 1)
            h = lanes >> B.bit_length() - 1
            cols = cls_v[...].at[j].get(mode="promise_in_bounds")
            row = jnp.minimum(j, half - 1) + half * h
            flat = row * S + cols
            idx_v[...] = jnp.where(lanes < nout, flat, 0)
            pltpu.async_copy(
                table_hbm.at[idx_v.at[pl.ds(0, nout)]], rows_v, sem
            ).wait()
            pltpu.sync_copy(rows_v, out_hbm)

    return gather_rows(table, cls16)


# speedup vs baseline: 4.3405x; 1.0714x over previous
import functools

import jax
import jax.numpy as jnp
from jax import lax
from jax.experimental import pallas as pl
from jax.experimental.pallas import tpu as pltpu
from jax.experimental.pallas import tpu_sc as plsc


def kernel(hidden_states, cls_index):
    B, S, H = hidden_states.shape
    half = B // 2
    nout = 2 * B
    table = hidden_states.reshape(B * S, H)
    cls16 = jnp.zeros((16,), jnp.int32).at[:B].set(cls_index)

    mesh = plsc.VectorSubcoreMesh(
        core_axis_name="c", subcore_axis_name="s", num_cores=1, num_subcores=1
    )

    @functools.partial(
        pl.kernel,
        out_type=jax.ShapeDtypeStruct((nout, H), jnp.float32),
        mesh=mesh,
        scratch_types=[
            pltpu.VMEM((16,), jnp.int32),
            pltpu.VMEM((16,), jnp.int32),
            pltpu.VMEM((nout, H), jnp.float32),
            pltpu.SemaphoreType.DMA,
        ],
    )
    def gather_rows(table_hbm, cls_hbm, out_hbm, cls_v, idx_v, rows_v, sem):
        pltpu.sync_copy(cls_hbm, cls_v)
        lanes = lax.iota(jnp.int32, 16)
        j = lanes & (B - 1)
        h = lanes >> B.bit_length() - 1
        cols = cls_v[...].at[j].get(mode="promise_in_bounds")
        row = jnp.minimum(j, half - 1) + half * h
        flat = row * S + cols
        idx_v[...] = jnp.where(lanes < nout, flat, 0)
        pltpu.async_copy(
            table_hbm.at[idx_v.at[pl.ds(0, nout)]], rows_v, sem
        ).wait()
        pltpu.sync_copy(rows_v, out_hbm)

    return gather_rows(table, cls16)


# speedup vs baseline: 4.5229x; 1.0420x over previous
import functools

import jax
import jax.numpy as jnp
from jax.experimental import pallas as pl
from jax.experimental.pallas import tpu as pltpu
from jax.experimental.pallas import tpu_sc as plsc


def kernel(hidden_states, cls_index):
    B, S, H = hidden_states.shape
    half = B // 2
    nout = 2 * B
    table = hidden_states.reshape(B * S, H)

    mesh = plsc.ScalarSubcoreMesh(axis_name="c", num_cores=1)

    @functools.partial(
        pl.kernel,
        out_type=jax.ShapeDtypeStruct((nout, H), jnp.float32),
        mesh=mesh,
        scratch_types=[
            pltpu.SMEM((B,), jnp.int32),
            pltpu.SemaphoreType.DMA,
        ],
    )
    def gather_rows(table_hbm, cls_hbm, out_hbm, cls_s, sem):
        pltpu.sync_copy(cls_hbm, cls_s)
        copies = []
        for b in range(nout):
            j = b % B
            row = min(j, half - 1) + half * (b // B)
            flat = row * S + cls_s[j]
            copies.append(
                pltpu.make_async_copy(
                    table_hbm.at[pl.ds(flat, 1)], out_hbm.at[pl.ds(b, 1)], sem
                )
            )
        for c in copies:
            c.start()
        for c in copies:
            c.wait()

    return gather_rows(table, cls_index)
